# baseline (speedup 1.0000x reference)
import functools

import jax
import jax.numpy as jnp
from jax import lax
from jax.experimental import pallas as pl
from jax.experimental.pallas import tpu as pltpu
from jax.experimental.pallas import tpu_sc as plsc

C = 96
H = 512
W = 512
HW = H * W
P = 16384

NC = 2
NS = 16
NW = NC * NS
PW = P // NW
BC = 128
NCHUNK = PW // BC
LANES = 16
CV = C // LANES


def _smooth_l1_acc(acc, va, vb):
    d = va - vb
    ad = jnp.abs(d)
    return acc + jnp.where(ad < 1.0, 0.5 * d * d, ad - 0.5)


_ROUNDS = (
    ((0, 1, 2, 3), ((0, 1), (2, 3))),
    ((4, 5), ((0, 1),)),
    ((6, 7, 8), ((0, 1), (1, 2), (0, 2))),
    ((9, 10, 11), ((0, 1), (0, 2))),
)


def _sc_body(ft, c0, c1, c2, c3, c4, c5, c6, c7, c8, c9, c10, c11,
             out, i0, i1, i2, i3, b0, b1, b2, b3, accv, sem):
    choices = (c0, c1, c2, c3, c4, c5, c6, c7, c8, c9, c10, c11)
    ibufs = (i0, i1, i2, i3)
    bufs = (b0, b1, b2, b3)
    wid = lax.axis_index("s") * NC + lax.axis_index("c")
    base = wid * PW

    acc = jnp.zeros((LANES,), jnp.float32)
    for arrays, pairs in _ROUNDS:
        for t in range(NCHUNK):
            off = base + t * BC
            for j, a in enumerate(arrays):
                pltpu.sync_copy(choices[a].at[pl.ds(off, BC)], ibufs[j])
            cps = [pltpu.async_copy(ft.at[ibufs[j]], bufs[j], sem)
                   for j in range(len(arrays))]
            for cp in cps:
                cp.wait()

            def row_body(r, a, pairs=pairs):
                for (x, y) in pairs:
                    for k in range(CV):
                        va = bufs[x][r, pl.ds(k * LANES, LANES)]
                        vb = bufs[y][r, pl.ds(k * LANES, LANES)]
                        a = _smooth_l1_acc(a, va, vb)
                return a

            acc = lax.fori_loop(0, BC, row_body, acc)

    accv[...] = acc
    pltpu.sync_copy(accv, out.at[wid])


def _sc_loss(ft, *choices):
    mesh = plsc.VectorSubcoreMesh(core_axis_name="c", subcore_axis_name="s")
    f = pl.kernel(
        _sc_body,
        out_type=jax.ShapeDtypeStruct((NW, LANES), jnp.float32),
        mesh=mesh,
        scratch_types=[
            pltpu.VMEM((BC,), jnp.int32),
            pltpu.VMEM((BC,), jnp.int32),
            pltpu.VMEM((BC,), jnp.int32),
            pltpu.VMEM((BC,), jnp.int32),
            pltpu.VMEM((BC, C), jnp.float32),
            pltpu.VMEM((BC, C), jnp.float32),
            pltpu.VMEM((BC, C), jnp.float32),
            pltpu.VMEM((BC, C), jnp.float32),
            pltpu.VMEM((LANES,), jnp.float32),
            pltpu.SemaphoreType.DMA,
        ],
        compiler_params=pltpu.CompilerParams(use_tc_tiling_on_sc=False),
    )
    return f(ft, *choices)


def kernel(features, vein_choice1, vein_choice2, artery_choice1,
           artery_choice2, background_choice1, background_choice2,
           background_choice_n, vein_choice_n, artery_choice_n,
           background_choice_nb, vein_choice_nb, artery_choice_nb):
    ft = features.reshape(C, HW).T
    parts = _sc_loss(
        ft,
        vein_choice1.astype(jnp.int32), artery_choice2.astype(jnp.int32),
        artery_choice1.astype(jnp.int32), vein_choice2.astype(jnp.int32),
        background_choice1.astype(jnp.int32),
        background_choice2.astype(jnp.int32),
        background_choice_n.astype(jnp.int32),
        vein_choice_n.astype(jnp.int32),
        artery_choice_n.astype(jnp.int32),
        background_choice_nb.astype(jnp.int32),
        vein_choice_nb.astype(jnp.int32),
        artery_choice_nb.astype(jnp.int32),
    )
    return jnp.sum(parts) / (C * P)


# speedup vs baseline: 1.0906x; 1.0906x over previous
import functools

import jax
import jax.numpy as jnp
from jax import lax
from jax.experimental import pallas as pl
from jax.experimental.pallas import tpu as pltpu
from jax.experimental.pallas import tpu_sc as plsc

C = 96
H = 512
W = 512
HW = H * W
P = 16384

NC = 2
NS = 16
NW = NC * NS
PW = P // NW
BC = 128
NCHUNK = PW // BC
LANES = 16
CV = C // LANES
ROWU = 2

_ROUNDS = (
    ((0, 1, 2, 3), ((0, 1), (2, 3))),
    ((4, 5), ((0, 1),)),
    ((6, 7, 8), ((0, 1), (1, 2), (0, 2))),
    ((9, 10, 11), ((0, 1), (0, 2))),
)
_STEPS = tuple((arrays, pairs, t)
               for arrays, pairs in _ROUNDS for t in range(NCHUNK))


def _smooth_l1_acc(acc, va, vb):
    d = va - vb
    q = (0.5 * d) * d
    l = jnp.abs(d) - 0.5
    return acc + jnp.minimum(q, l)


def _sc_body(ft, c0, c1, c2, c3, c4, c5, c6, c7, c8, c9, c10, c11,
             out, i0, i1, i2, i3, i4, i5, i6, i7,
             b0, b1, b2, b3, b4, b5, b6, b7, accv, sem0, sem1):
    choices = (c0, c1, c2, c3, c4, c5, c6, c7, c8, c9, c10, c11)
    ibufs = ((i0, i1, i2, i3), (i4, i5, i6, i7))
    bufs = ((b0, b1, b2, b3), (b4, b5, b6, b7))
    sems = (sem0, sem1)
    wid = lax.axis_index("s") * NC + lax.axis_index("c")
    base = wid * PW

    def issue(step_idx):
        arrays, _, t = _STEPS[step_idx]
        s = step_idx % 2
        off = base + t * BC
        for j, a in enumerate(arrays):
            pltpu.sync_copy(choices[a].at[pl.ds(off, BC)], ibufs[s][j])
        return [pltpu.async_copy(ft.at[ibufs[s][j]], bufs[s][j], sems[s])
                for j in range(len(arrays))]

    acc = jnp.zeros((LANES,), jnp.float32)
    pending = issue(0)
    for i, (arrays, pairs, t) in enumerate(_STEPS):
        s = i % 2
        for cp in pending:
            cp.wait()
        if i + 1 < len(_STEPS):
            pending = issue(i + 1)

        def row_body(r, a, pairs=pairs, arrays=arrays, s=s):
            for u in range(ROWU):
                rr = r * ROWU + u
                for k in range(CV):
                    vs = [bufs[s][j][rr, pl.ds(k * LANES, LANES)]
                          for j in range(len(arrays))]
                    for (x, y) in pairs:
                        a = _smooth_l1_acc(a, vs[x], vs[y])
            return a

        acc = lax.fori_loop(0, BC // ROWU, row_body, acc)

    accv[...] = acc
    pltpu.sync_copy(accv, out.at[wid])


def _sc_loss(ft, *choices):
    mesh = plsc.VectorSubcoreMesh(core_axis_name="c", subcore_axis_name="s")
    f = pl.kernel(
        _sc_body,
        out_type=jax.ShapeDtypeStruct((NW, LANES), jnp.float32),
        mesh=mesh,
        scratch_types=(
            [pltpu.VMEM((BC,), jnp.int32) for _ in range(8)]
            + [pltpu.VMEM((BC, C), jnp.float32) for _ in range(8)]
            + [pltpu.VMEM((LANES,), jnp.float32),
               pltpu.SemaphoreType.DMA, pltpu.SemaphoreType.DMA]
        ),
        compiler_params=pltpu.CompilerParams(use_tc_tiling_on_sc=False),
    )
    return f(ft, *choices)


def kernel(features, vein_choice1, vein_choice2, artery_choice1,
           artery_choice2, background_choice1, background_choice2,
           background_choice_n, vein_choice_n, artery_choice_n,
           background_choice_nb, vein_choice_nb, artery_choice_nb):
    ft = features.reshape(C, HW).T
    parts = _sc_loss(
        ft,
        vein_choice1.astype(jnp.int32), artery_choice2.astype(jnp.int32),
        artery_choice1.astype(jnp.int32), vein_choice2.astype(jnp.int32),
        background_choice1.astype(jnp.int32),
        background_choice2.astype(jnp.int32),
        background_choice_n.astype(jnp.int32),
        vein_choice_n.astype(jnp.int32),
        artery_choice_n.astype(jnp.int32),
        background_choice_nb.astype(jnp.int32),
        vein_choice_nb.astype(jnp.int32),
        artery_choice_nb.astype(jnp.int32),
    )
    return jnp.sum(parts) / (C * P)
